# baseline (speedup 1.0000x reference)
import jax
import jax.numpy as jnp
from jax import lax
from jax.experimental import pallas as pl
from jax.experimental.pallas import tpu as pltpu
from jax.experimental.pallas import tpu_sc as plsc

_B, _D, _C = 1024, 32, 100000
_DD = _D * _D
_TSIM = 0.8
_EPS = 1e-8
_TILE = 2048
_NT = 49
_CPAD = _NT * _TILE
_NEG = -3.0e38
_TW = 1024
_C0 = 51200
_TILE_OFF = _C0 // _TW
_NTW = 98 - _TILE_OFF
_NW = 32
_GPW = 4
_SCCH = 8
_CLEN = _C0 // _SCCH
_NG = _B // 16


def _phase_a(mv_ref, pat_ref, usage_ref, sc_ref,
             idx_ref, sim_ref, wcol_ref, wcoli_ref, nuse_ref, pay_ref,
             load_ref, outer_ref,
             bval, bidx, bsim, blru, bsc, nnz):
  pid = pl.program_id(0)

  @pl.when(pid == 0)
  def _init():
    bval[...] = jnp.full((_B, 1), _NEG, jnp.float32)
    bidx[...] = jnp.zeros((_B, 1), jnp.float32)
    bsim[...] = jnp.zeros((_B, 1), jnp.float32)
    blru[...] = jnp.zeros((_B, 1), jnp.float32)
    bsc[...] = jnp.zeros((_B, 1), jnp.float32)
    nnz[...] = jnp.zeros((1, 1), jnp.float32)

  mv = mv_ref[...]
  vn = mv / jnp.maximum(
      jnp.sqrt(jnp.sum(mv * mv, axis=1, keepdims=True)), _EPS)
  p = pat_ref[...]

  coli1 = lax.broadcasted_iota(jnp.int32, (1, _TILE), 1)
  valid1 = pid * _TILE + coli1 < _C

  pn = jnp.where(
      valid1,
      p / jnp.maximum(jnp.sqrt(jnp.sum(p * p, axis=0, keepdims=True)), _EPS),
      0.0)
  sim = lax.dot_general(vn, pn, (((1,), (0,)), ((), ())),
                        preferred_element_type=jnp.float32)

  usage = usage_ref[0]
  lru01 = (1.0 / (1.0 + usage)) * 0.1
  scv = sc_ref[0]

  selrow = jnp.where(valid1, lru01, _NEG)
  mrow = jnp.max(selrow, axis=1, keepdims=True)
  jrow = jnp.min(jnp.where(selrow == mrow, coli1, 2 ** 30),
                 axis=1, keepdims=True)
  atrow = coli1 == jrow
  lru_r = jnp.max(jnp.where(atrow, lru01, _NEG), axis=1, keepdims=True)
  sc_r = jnp.max(jnp.where(atrow, scv, _NEG), axis=1, keepdims=True)
  simmax = jnp.max(sim)

  anyupd = jnp.max(selrow) > jnp.min(bval[...])

  @pl.when((simmax < _TSIM) & anyupd)
  def _fast():
    coli = lax.broadcasted_iota(jnp.int32, (_B, _TILE), 1)
    sim_at = jnp.max(jnp.where(coli == jrow, sim, _NEG),
                     axis=1, keepdims=True)
    upd = mrow > bval[...]
    gidxf = (pid * _TILE + jrow).astype(jnp.float32)
    bval[...] = jnp.where(upd, mrow, bval[...])
    bidx[...] = jnp.where(upd, gidxf, bidx[...])
    bsim[...] = jnp.where(upd, sim_at, bsim[...])
    blru[...] = jnp.where(upd, lru_r, blru[...])
    bsc[...] = jnp.where(upd, sc_r, bsc[...])

  @pl.when(simmax >= _TSIM)
  def _slow():
    coli = lax.broadcasted_iota(jnp.int32, (_B, _TILE), 1)
    valid = pid * _TILE + coli < _C
    lru_b = jnp.broadcast_to(lru01, (_B, _TILE))
    sel = jnp.where(sim - _TSIM < 0, lru_b, lru_b - 1e9)
    sel = jnp.where(valid, sel, _NEG)

    m = jnp.max(sel, axis=1, keepdims=True)
    jloc = jnp.min(jnp.where(sel == m, coli, 2 ** 30), axis=1, keepdims=True)
    at = coli == jloc
    sim_at = jnp.max(jnp.where(at, sim, _NEG), axis=1, keepdims=True)
    lru_at = jnp.max(jnp.where(at, lru_b, _NEG), axis=1, keepdims=True)
    sc_at = jnp.max(jnp.where(at, jnp.broadcast_to(scv, (_B, _TILE)), _NEG),
                    axis=1, keepdims=True)

    upd = m > bval[...]
    bval[...] = jnp.where(upd, m, bval[...])
    bidx[...] = jnp.where(upd, (pid * _TILE + jloc).astype(jnp.float32),
                          bidx[...])
    bsim[...] = jnp.where(upd, sim_at, bsim[...])
    blru[...] = jnp.where(upd, lru_at, blru[...])
    bsc[...] = jnp.where(upd, sc_at, bsc[...])

  nnz[...] += jnp.sum(jnp.where(valid1 & (usage > 0), 1.0, 0.0),
                      axis=(0, 1), keepdims=True)

  @pl.when(pid == _NT - 1)
  def _fin():
    idxf = bidx[...]
    idx_ref[...] = idxf.astype(jnp.int32)
    sim_ref[...] = bsim[...]
    usage_at = 0.1 / blru[...] - 1.0

    ii = lax.broadcasted_iota(jnp.int32, (_B, _B), 0)
    jj = lax.broadcasted_iota(jnp.int32, (_B, _B), 1)
    eyef = jnp.where(ii == jj, 1.0, 0.0)
    idx_row = lax.dot_general(idxf, eyef, (((0,), (0,)), ((), ())),
                              preferred_element_type=jnp.float32,
                              precision=lax.Precision.HIGHEST)
    eqm = idxf == idx_row
    winf = jnp.max(jnp.where(eqm, jj, -1), axis=1, keepdims=True)
    cnt = jnp.sum(jnp.where(eqm, 1.0, 0.0), axis=1, keepdims=True)
    nuse_ref[...] = usage_at + cnt

    own = lax.broadcasted_iota(jnp.int32, (_B, 1), 0)
    is_win = winf == own
    wcol = jnp.where(is_win, idxf, -1.0)
    wcol_ref[...] = wcol
    wcoli_ref[...] = wcol.astype(jnp.int32)

    uniq = jnp.sum(jnp.where(is_win, 1.0, 0.0), axis=(0, 1), keepdims=True)
    was_nz = jnp.sum(jnp.where(is_win & (usage_at > 0), 1.0, 0.0),
                     axis=(0, 1), keepdims=True)
    load_ref[...] = (nnz[...] - was_nz + uniq) / _C

    mvT = lax.dot_general(
        jnp.where(lax.broadcasted_iota(jnp.int32, (_D, _D), 0) ==
                  lax.broadcasted_iota(jnp.int32, (_D, _D), 1), 1.0, 0.0),
        mv, (((1,), (1,)), ((), ())),
        preferred_element_type=jnp.float32,
        precision=lax.Precision.HIGHEST)
    sumsq = jnp.sum(mv * mv, axis=1, keepdims=True)
    gate = 1.0 / (1.0 + jnp.exp(-sumsq))
    gateT = lax.dot_general(gate, eyef, (((0,), (0,)), ((), ())),
                            preferred_element_type=jnp.float32,
                            precision=lax.Precision.HIGHEST)
    bscT = lax.dot_general(bsc[...], eyef, (((0,), (0,)), ((), ())),
                           preferred_element_type=jnp.float32,
                           precision=lax.Precision.HIGHEST)
    pay_ref[0:_D, :] = mvT
    pay_ref[_D:2 * _D, :] = jnp.broadcast_to(gateT, (_D, _B))
    for d in range(_D):
      outer_ref[d * _D:(d + 1) * _D, :] = (mvT * mvT[d:d + 1, :]) * bscT


_PHASE_A_KWARGS = dict(
    grid=(_NT,),
    in_specs=[
        pl.BlockSpec((_B, _D), lambda i: (0, 0)),
        pl.BlockSpec((_D, _TILE), lambda i: (0, i)),
        pl.BlockSpec((1, 1, _TILE), lambda i: (i, 0, 0)),
        pl.BlockSpec((1, 1, _TILE), lambda i: (i, 0, 0)),
    ],
    out_specs=[
        pl.BlockSpec((_B, 1), lambda i: (0, 0)),
        pl.BlockSpec((_B, 1), lambda i: (0, 0)),
        pl.BlockSpec((_B, 1), lambda i: (0, 0)),
        pl.BlockSpec((_B, 1), lambda i: (0, 0)),
        pl.BlockSpec((_B, 1), lambda i: (0, 0)),
        pl.BlockSpec((2 * _D, _B), lambda i: (0, 0)),
        pl.BlockSpec((1, 1), lambda i: (0, 0)),
        pl.BlockSpec((_DD, _B), lambda i: (0, 0)),
    ],
    out_shape=[
        jax.ShapeDtypeStruct((_B, 1), jnp.int32),
        jax.ShapeDtypeStruct((_B, 1), jnp.float32),
        jax.ShapeDtypeStruct((_B, 1), jnp.float32),
        jax.ShapeDtypeStruct((_B, 1), jnp.int32),
        jax.ShapeDtypeStruct((_B, 1), jnp.float32),
        jax.ShapeDtypeStruct((2 * _D, _B), jnp.float32),
        jax.ShapeDtypeStruct((1, 1), jnp.float32),
        jax.ShapeDtypeStruct((_DD, _B), jnp.float32),
    ],
    scratch_shapes=[pltpu.VMEM((_B, 1), jnp.float32)] * 5
    + [pltpu.VMEM((1, 1), jnp.float32)],
    compiler_params=pltpu.CompilerParams(
        dimension_semantics=("arbitrary",)),
)


def _phase_b(pat_ref, gate_ref, usage_ref, wcol_ref, nusec_ref, pay_ref,
             npat_ref, ngate_ref, nuse_ref):
  pid = pl.program_id(0)
  wcol = wcol_ref[...]
  lo = (pid * _TILE).astype(jnp.float32)
  hi = lo + float(_TILE)
  nhit = jnp.sum(jnp.where((wcol >= lo) & (wcol < hi), 1.0, 0.0))

  @pl.when(nhit == 0.0)
  def _copy():
    npat_ref[...] = pat_ref[...]
    ngate_ref[...] = gate_ref[...]
    nuse_ref[0] = usage_ref[0]

  @pl.when(nhit > 0.0)
  def _merge():
    rowr = pid * _TILE + lax.broadcasted_iota(jnp.int32, (1, _TILE), 1)
    oh2 = jnp.where(wcol == rowr.astype(jnp.float32), 1.0, 0.0)
    writ2 = jnp.sum(oh2, axis=0, keepdims=True) > 0.0
    gathered = lax.dot_general(pay_ref[...], oh2, (((1,), (0,)), ((), ())),
                               preferred_element_type=jnp.float32,
                               precision=lax.Precision.HIGHEST)
    npat_ref[...] = jnp.where(writ2, gathered[0:_D, :], pat_ref[...])
    ngate_ref[...] = jnp.where(writ2, gathered[_D:2 * _D, :], gate_ref[...])
    nuse_row = lax.dot_general(nusec_ref[...], oh2, (((0,), (0,)), ((), ())),
                               preferred_element_type=jnp.float32,
                               precision=lax.Precision.HIGHEST)
    nuse_ref[0] = jnp.where(writ2, nuse_row, usage_ref[0])


_PHASE_B_KWARGS = dict(
    grid=(_NT,),
    in_specs=[
        pl.BlockSpec((_D, _TILE), lambda i: (0, i)),
        pl.BlockSpec((_D, _TILE), lambda i: (0, i)),
        pl.BlockSpec((1, 1, _TILE), lambda i: (i, 0, 0)),
        pl.BlockSpec((_B, 1), lambda i: (0, 0)),
        pl.BlockSpec((_B, 1), lambda i: (0, 0)),
        pl.BlockSpec((2 * _D, _B), lambda i: (0, 0)),
    ],
    out_specs=[
        pl.BlockSpec((_D, _TILE), lambda i: (0, i)),
        pl.BlockSpec((_D, _TILE), lambda i: (0, i)),
        pl.BlockSpec((1, 1, _TILE), lambda i: (i, 0, 0)),
    ],
    out_shape=[
        jax.ShapeDtypeStruct((_D, _C), jnp.float32),
        jax.ShapeDtypeStruct((_D, _C), jnp.float32),
        jax.ShapeDtypeStruct((_NT, 1, _TILE), jnp.float32),
    ],
    compiler_params=pltpu.CompilerParams(
        dimension_semantics=("arbitrary",)),
)


def _phase_w(sw_ref, wcol_ref, outer_ref, nsw_ref):
  pid = pl.program_id(0) + _TILE_OFF
  wcol = wcol_ref[...]
  lo = (pid * _TW).astype(jnp.float32)
  hi = lo + float(_TW)
  nhit = jnp.sum(jnp.where((wcol >= lo) & (wcol < hi), 1.0, 0.0))

  @pl.when(nhit == 0.0)
  def _copy():
    nsw_ref[...] = sw_ref[...]

  @pl.when(nhit > 0.0)
  def _merge():
    rowr = pid * _TW + lax.broadcasted_iota(jnp.int32, (1, _TW), 1)
    oh2 = jnp.where(wcol == rowr.astype(jnp.float32), 1.0, 0.0)
    writ2 = jnp.sum(oh2, axis=0, keepdims=True) > 0.0
    gathered = lax.dot_general(outer_ref[...], oh2, (((1,), (0,)), ((), ())),
                               preferred_element_type=jnp.float32,
                               precision=lax.Precision.HIGHEST)
    nsw_ref[...] = jnp.where(writ2, gathered, sw_ref[...])


_PHASE_W_KWARGS = dict(
    grid=(_NTW,),
    in_specs=[
        pl.BlockSpec((_DD, _TW), lambda i: (0, i + _TILE_OFF)),
        pl.BlockSpec((_B, 1), lambda i: (0, 0)),
        pl.BlockSpec((_DD, _B), lambda i: (0, 0)),
    ],
    out_specs=[pl.BlockSpec((_DD, _TW), lambda i: (0, i + _TILE_OFF))],
    out_shape=[jax.ShapeDtypeStruct((_DD, _C), jnp.float32)],
    compiler_params=pltpu.CompilerParams(
        dimension_semantics=("arbitrary",)),
)


def _sc_w(sw_hbm, wcoli_hbm, outer_hbm, nsw_in, nsw_out,
          idxv, owb, sbuf0, sbuf1, semg, sems):
  del nsw_in
  wid = lax.axis_index("s") * 2 + lax.axis_index("c")
  pltpu.sync_copy(wcoli_hbm, idxv)

  def _strip(g8, k):
    off = pl.multiple_of(k * _CLEN, 128)
    return sw_hbm.at[pl.ds(g8, 8), pl.ds(off, _CLEN)]

  def _ostrip(g8, k):
    off = pl.multiple_of(k * _CLEN, 128)
    return nsw_out.at[pl.ds(g8, 8), pl.ds(off, _CLEN)]

  def _fixups(buf, k):
    base = k * _CLEN
    for dd in range(8):
      ddv = jnp.full((16,), dd, jnp.int32)
      for g in range(_NG):
        ig = idxv[pl.ds(g * 16, 16)]
        ps = ig - base
        m = (ps >= 0) & (ps < _CLEN)
        vals = owb[dd, pl.ds(g * 16, 16)]
        plsc.store_scatter(buf, [ddv, ps], vals, mask=m)

  @pl.loop(0, _GPW)
  def _grp(gi):
    g8 = pl.multiple_of((wid * _GPW + gi) * 8, 8)
    pltpu.sync_copy(outer_hbm.at[pl.ds(g8, 8), :], owb)
    pltpu.async_copy(_strip(g8, 0), sbuf0, semg)
    pltpu.async_copy(_strip(g8, 1), sbuf1, semg)

    @pl.loop(0, _SCCH // 2)
    def _pair(h):
      k0 = 2 * h
      k1 = 2 * h + 1
      pltpu.make_async_copy(_strip(g8, k0), sbuf0, semg).wait()
      _fixups(sbuf0, k0)
      pltpu.async_copy(sbuf0, _ostrip(g8, k0), sems)
      pltpu.make_async_copy(_strip(g8, k1), sbuf1, semg).wait()
      _fixups(sbuf1, k1)
      pltpu.async_copy(sbuf1, _ostrip(g8, k1), sems)
      pltpu.make_async_copy(sbuf0, _ostrip(g8, k0), sems).wait()
      pltpu.make_async_copy(sbuf1, _ostrip(g8, k1), sems).wait()

      @pl.when(h + 1 < _SCCH // 2)
      def _nextgathers():
        pltpu.async_copy(_strip(g8, k0 + 2), sbuf0, semg)
        pltpu.async_copy(_strip(g8, k1 + 2), sbuf1, semg)


_sc_w_cache = []


def _get_sc_w():
  if not _sc_w_cache:
    from jax._src.pallas import mpmd as _mpmd
    mesh = plsc.VectorSubcoreMesh(core_axis_name="c", subcore_axis_name="s")
    _sc_w_cache.append(_mpmd._mpmd_map(
        [(mesh, _sc_w)],
        out_types=[jax.ShapeDtypeStruct((_DD, _C), jnp.float32)],
        input_output_aliases={3: 0},
        compiler_params=pltpu.CompilerParams(needs_layout_passes=False),
        scratch_types=[
            pltpu.VMEM((_B,), jnp.int32),
            pltpu.VMEM((8, _B), jnp.float32),
            pltpu.VMEM((8, _CLEN), jnp.float32),
            pltpu.VMEM((8, _CLEN), jnp.float32),
            pltpu.SemaphoreType.DMA,
            pltpu.SemaphoreType.DMA,
        ],
    ))
  return _sc_w_cache[0]


def kernel(memory_vector, memory_patterns, synaptic_weights, synaptic_gates,
           structural_complexity, usage_counts):
  pad = _CPAD - _C
  usage_p = jnp.pad(usage_counts, (0, pad)).reshape(_NT, 1, _TILE)
  sc_p = jnp.pad(structural_complexity, (0, pad)).reshape(_NT, 1, _TILE)

  patT = memory_patterns.T
  gateT = synaptic_gates.T
  swT = synaptic_weights.reshape(_C, _DD).T

  (idx2, sims, wcol, wcoli, nusec, payT, load2, outersT) = (
      pl.pallas_call(_phase_a, **_PHASE_A_KWARGS)(
          memory_vector, patT, usage_p, sc_p))

  (nswT_part,) = pl.pallas_call(_phase_w, **_PHASE_W_KWARGS)(
      swT, wcol, outersT)

  (nswT,) = _get_sc_w()(swT, wcoli.reshape(_B), outersT, nswT_part)

  npatT, ngateT, nuse_p = pl.pallas_call(_phase_b, **_PHASE_B_KWARGS)(
      patT, gateT, usage_p, wcol, nusec, payT)

  return (idx2.reshape(_B), sims, npatT.T,
          nswT.T.reshape(_C, _D, _D), ngateT.T,
          nuse_p.reshape(_CPAD)[:_C], load2.reshape(()))


# speedup vs baseline: 1.2943x; 1.2943x over previous
import jax
import jax.numpy as jnp
from jax import lax
from jax.experimental import pallas as pl
from jax.experimental.pallas import tpu as pltpu

_B, _D, _C = 1024, 32, 100000
_DD = _D * _D
_TSIM = 0.8
_EPS = 1e-8
_TILE = 2048
_NT = 49
_CPAD = _NT * _TILE
_NEG = -3.0e38
_TW = 1024
_NTW = 98


def _phase_a(mv_ref, pat_ref, usage_ref, sc_ref,
             idx_ref, sim_ref, wcol_ref, nuse_ref, pay_ref, load_ref,
             outer_ref,
             bval, bidx, bsim, blru, bsc, nnz):
  pid = pl.program_id(0)

  @pl.when(pid == 0)
  def _init():
    bval[...] = jnp.full((_B, 1), _NEG, jnp.float32)
    bidx[...] = jnp.zeros((_B, 1), jnp.float32)
    bsim[...] = jnp.zeros((_B, 1), jnp.float32)
    blru[...] = jnp.zeros((_B, 1), jnp.float32)
    bsc[...] = jnp.zeros((_B, 1), jnp.float32)
    nnz[...] = jnp.zeros((1, 1), jnp.float32)

  mv = mv_ref[...]
  vn = mv / jnp.maximum(
      jnp.sqrt(jnp.sum(mv * mv, axis=1, keepdims=True)), _EPS)
  p = pat_ref[...]

  coli1 = lax.broadcasted_iota(jnp.int32, (1, _TILE), 1)
  valid1 = pid * _TILE + coli1 < _C

  pn = jnp.where(
      valid1,
      p / jnp.maximum(jnp.sqrt(jnp.sum(p * p, axis=0, keepdims=True)), _EPS),
      0.0)
  sim = lax.dot_general(vn, pn, (((1,), (0,)), ((), ())),
                        preferred_element_type=jnp.float32)

  usage = usage_ref[0]
  lru01 = (1.0 / (1.0 + usage)) * 0.1
  scv = sc_ref[0]

  selrow = jnp.where(valid1, lru01, _NEG)
  mrow = jnp.max(selrow, axis=1, keepdims=True)
  jrow = jnp.min(jnp.where(selrow == mrow, coli1, 2 ** 30),
                 axis=1, keepdims=True)
  atrow = coli1 == jrow
  lru_r = jnp.max(jnp.where(atrow, lru01, _NEG), axis=1, keepdims=True)
  sc_r = jnp.max(jnp.where(atrow, scv, _NEG), axis=1, keepdims=True)
  simmax = jnp.max(sim)

  anyupd = jnp.max(selrow) > jnp.min(bval[...])

  @pl.when((simmax < _TSIM) & anyupd)
  def _fast():
    coli = lax.broadcasted_iota(jnp.int32, (_B, _TILE), 1)
    sim_at = jnp.max(jnp.where(coli == jrow, sim, _NEG),
                     axis=1, keepdims=True)
    upd = mrow > bval[...]
    gidxf = (pid * _TILE + jrow).astype(jnp.float32)
    bval[...] = jnp.where(upd, mrow, bval[...])
    bidx[...] = jnp.where(upd, gidxf, bidx[...])
    bsim[...] = jnp.where(upd, sim_at, bsim[...])
    blru[...] = jnp.where(upd, lru_r, blru[...])
    bsc[...] = jnp.where(upd, sc_r, bsc[...])

  @pl.when(simmax >= _TSIM)
  def _slow():
    coli = lax.broadcasted_iota(jnp.int32, (_B, _TILE), 1)
    valid = pid * _TILE + coli < _C
    lru_b = jnp.broadcast_to(lru01, (_B, _TILE))
    sel = jnp.where(sim - _TSIM < 0, lru_b, lru_b - 1e9)
    sel = jnp.where(valid, sel, _NEG)

    m = jnp.max(sel, axis=1, keepdims=True)
    jloc = jnp.min(jnp.where(sel == m, coli, 2 ** 30), axis=1, keepdims=True)
    at = coli == jloc
    sim_at = jnp.max(jnp.where(at, sim, _NEG), axis=1, keepdims=True)
    lru_at = jnp.max(jnp.where(at, lru_b, _NEG), axis=1, keepdims=True)
    sc_at = jnp.max(jnp.where(at, jnp.broadcast_to(scv, (_B, _TILE)), _NEG),
                    axis=1, keepdims=True)

    upd = m > bval[...]
    bval[...] = jnp.where(upd, m, bval[...])
    bidx[...] = jnp.where(upd, (pid * _TILE + jloc).astype(jnp.float32),
                          bidx[...])
    bsim[...] = jnp.where(upd, sim_at, bsim[...])
    blru[...] = jnp.where(upd, lru_at, blru[...])
    bsc[...] = jnp.where(upd, sc_at, bsc[...])

  nnz[...] += jnp.sum(jnp.where(valid1 & (usage > 0), 1.0, 0.0),
                      axis=(0, 1), keepdims=True)

  @pl.when(pid == _NT - 1)
  def _fin():
    idxf = bidx[...]
    idx_ref[...] = idxf.astype(jnp.int32)
    sim_ref[...] = bsim[...]
    usage_at = 0.1 / blru[...] - 1.0

    ii = lax.broadcasted_iota(jnp.int32, (_B, _B), 0)
    jj = lax.broadcasted_iota(jnp.int32, (_B, _B), 1)
    eyef = jnp.where(ii == jj, 1.0, 0.0)
    idx_row = lax.dot_general(idxf, eyef, (((0,), (0,)), ((), ())),
                              preferred_element_type=jnp.float32,
                              precision=lax.Precision.HIGHEST)
    eqm = idxf == idx_row
    winf = jnp.max(jnp.where(eqm, jj, -1), axis=1, keepdims=True)
    cnt = jnp.sum(jnp.where(eqm, 1.0, 0.0), axis=1, keepdims=True)
    nuse_ref[...] = usage_at + cnt

    own = lax.broadcasted_iota(jnp.int32, (_B, 1), 0)
    is_win = winf == own
    wcol = jnp.where(is_win, idxf, -1.0)
    wcol_ref[...] = wcol

    uniq = jnp.sum(jnp.where(is_win, 1.0, 0.0), axis=(0, 1), keepdims=True)
    was_nz = jnp.sum(jnp.where(is_win & (usage_at > 0), 1.0, 0.0),
                     axis=(0, 1), keepdims=True)
    load_ref[...] = (nnz[...] - was_nz + uniq) / _C

    mvT = lax.dot_general(
        jnp.where(lax.broadcasted_iota(jnp.int32, (_D, _D), 0) ==
                  lax.broadcasted_iota(jnp.int32, (_D, _D), 1), 1.0, 0.0),
        mv, (((1,), (1,)), ((), ())),
        preferred_element_type=jnp.float32,
        precision=lax.Precision.HIGHEST)
    sumsq = jnp.sum(mv * mv, axis=1, keepdims=True)
    gate = 1.0 / (1.0 + jnp.exp(-sumsq))
    gateT = lax.dot_general(gate, eyef, (((0,), (0,)), ((), ())),
                            preferred_element_type=jnp.float32,
                            precision=lax.Precision.HIGHEST)
    bscT = lax.dot_general(bsc[...], eyef, (((0,), (0,)), ((), ())),
                           preferred_element_type=jnp.float32,
                           precision=lax.Precision.HIGHEST)
    pay_ref[0:_D, :] = mvT
    pay_ref[_D:2 * _D, :] = jnp.broadcast_to(gateT, (_D, _B))
    for d in range(_D):
      outer_ref[d * _D:(d + 1) * _D, :] = (mvT * mvT[d:d + 1, :]) * bscT


_PHASE_A_KWARGS = dict(
    grid=(_NT,),
    in_specs=[
        pl.BlockSpec((_B, _D), lambda i: (0, 0)),
        pl.BlockSpec((_D, _TILE), lambda i: (0, i)),
        pl.BlockSpec((1, 1, _TILE), lambda i: (i, 0, 0)),
        pl.BlockSpec((1, 1, _TILE), lambda i: (i, 0, 0)),
    ],
    out_specs=[
        pl.BlockSpec((_B, 1), lambda i: (0, 0)),
        pl.BlockSpec((_B, 1), lambda i: (0, 0)),
        pl.BlockSpec((_B, 1), lambda i: (0, 0)),
        pl.BlockSpec((_B, 1), lambda i: (0, 0)),
        pl.BlockSpec((2 * _D, _B), lambda i: (0, 0)),
        pl.BlockSpec((1, 1), lambda i: (0, 0)),
        pl.BlockSpec((_DD, _B), lambda i: (0, 0)),
    ],
    out_shape=[
        jax.ShapeDtypeStruct((_B, 1), jnp.int32),
        jax.ShapeDtypeStruct((_B, 1), jnp.float32),
        jax.ShapeDtypeStruct((_B, 1), jnp.float32),
        jax.ShapeDtypeStruct((_B, 1), jnp.float32),
        jax.ShapeDtypeStruct((2 * _D, _B), jnp.float32),
        jax.ShapeDtypeStruct((1, 1), jnp.float32),
        jax.ShapeDtypeStruct((_DD, _B), jnp.float32),
    ],
    scratch_shapes=[pltpu.VMEM((_B, 1), jnp.float32)] * 5
    + [pltpu.VMEM((1, 1), jnp.float32)],
    compiler_params=pltpu.CompilerParams(
        dimension_semantics=("arbitrary",)),
)


def _phase_b(pat_ref, gate_ref, usage_ref, wcol_ref, nusec_ref, pay_ref,
             npat_ref, ngate_ref, nuse_ref):
  pid = pl.program_id(0)
  wcol = wcol_ref[...]
  lo = (pid * _TILE).astype(jnp.float32)
  hi = lo + float(_TILE)
  nhit = jnp.sum(jnp.where((wcol >= lo) & (wcol < hi), 1.0, 0.0))

  @pl.when(nhit == 0.0)
  def _copy():
    npat_ref[...] = pat_ref[...]
    ngate_ref[...] = gate_ref[...]
    nuse_ref[0] = usage_ref[0]

  @pl.when(nhit > 0.0)
  def _merge():
    rowr = pid * _TILE + lax.broadcasted_iota(jnp.int32, (1, _TILE), 1)
    oh2 = jnp.where(wcol == rowr.astype(jnp.float32), 1.0, 0.0)
    writ2 = jnp.sum(oh2, axis=0, keepdims=True) > 0.0
    gathered = lax.dot_general(pay_ref[...], oh2, (((1,), (0,)), ((), ())),
                               preferred_element_type=jnp.float32,
                               precision=lax.Precision.HIGHEST)
    npat_ref[...] = jnp.where(writ2, gathered[0:_D, :], pat_ref[...])
    ngate_ref[...] = jnp.where(writ2, gathered[_D:2 * _D, :], gate_ref[...])
    nuse_row = lax.dot_general(nusec_ref[...], oh2, (((0,), (0,)), ((), ())),
                               preferred_element_type=jnp.float32,
                               precision=lax.Precision.HIGHEST)
    nuse_ref[0] = jnp.where(writ2, nuse_row, usage_ref[0])


_PHASE_B_KWARGS = dict(
    grid=(_NT,),
    in_specs=[
        pl.BlockSpec((_D, _TILE), lambda i: (0, i)),
        pl.BlockSpec((_D, _TILE), lambda i: (0, i)),
        pl.BlockSpec((1, 1, _TILE), lambda i: (i, 0, 0)),
        pl.BlockSpec((_B, 1), lambda i: (0, 0)),
        pl.BlockSpec((_B, 1), lambda i: (0, 0)),
        pl.BlockSpec((2 * _D, _B), lambda i: (0, 0)),
    ],
    out_specs=[
        pl.BlockSpec((_D, _TILE), lambda i: (0, i)),
        pl.BlockSpec((_D, _TILE), lambda i: (0, i)),
        pl.BlockSpec((1, 1, _TILE), lambda i: (i, 0, 0)),
    ],
    out_shape=[
        jax.ShapeDtypeStruct((_D, _C), jnp.float32),
        jax.ShapeDtypeStruct((_D, _C), jnp.float32),
        jax.ShapeDtypeStruct((_NT, 1, _TILE), jnp.float32),
    ],
    compiler_params=pltpu.CompilerParams(
        dimension_semantics=("arbitrary",)),
)


def _phase_w(sw_ref, wcol_ref, outer_ref, nsw_ref):
  pid = pl.program_id(0)
  wcol = wcol_ref[...]
  lo = (pid * _TW).astype(jnp.float32)
  hi = lo + float(_TW)
  nhit = jnp.sum(jnp.where((wcol >= lo) & (wcol < hi), 1.0, 0.0))

  @pl.when(nhit == 0.0)
  def _copy():
    nsw_ref[...] = sw_ref[...]

  @pl.when(nhit > 0.0)
  def _merge():
    rowr = pid * _TW + lax.broadcasted_iota(jnp.int32, (1, _TW), 1)
    oh2 = jnp.where(wcol == rowr.astype(jnp.float32), 1.0, 0.0)
    writ2 = jnp.sum(oh2, axis=0, keepdims=True) > 0.0
    gathered = lax.dot_general(outer_ref[...], oh2, (((1,), (0,)), ((), ())),
                               preferred_element_type=jnp.float32,
                               precision=lax.Precision.HIGHEST)
    nsw_ref[...] = jnp.where(writ2, gathered, sw_ref[...])


_PHASE_W_KWARGS = dict(
    grid=(_NTW,),
    in_specs=[
        pl.BlockSpec((_DD, _TW), lambda i: (0, i)),
        pl.BlockSpec((_B, 1), lambda i: (0, 0)),
        pl.BlockSpec((_DD, _B), lambda i: (0, 0)),
    ],
    out_specs=[pl.BlockSpec((_DD, _TW), lambda i: (0, i))],
    out_shape=[jax.ShapeDtypeStruct((_DD, _C), jnp.float32)],
    compiler_params=pltpu.CompilerParams(
        dimension_semantics=("arbitrary",)),
)


def kernel(memory_vector, memory_patterns, synaptic_weights, synaptic_gates,
           structural_complexity, usage_counts):
  pad = _CPAD - _C
  usage_p = jnp.pad(usage_counts, (0, pad)).reshape(_NT, 1, _TILE)
  sc_p = jnp.pad(structural_complexity, (0, pad)).reshape(_NT, 1, _TILE)

  patT = memory_patterns.T
  gateT = synaptic_gates.T
  swT = synaptic_weights.reshape(_C, _DD).T

  (idx2, sims, wcol, nusec, payT, load2, outersT) = (
      pl.pallas_call(_phase_a, **_PHASE_A_KWARGS)(
          memory_vector, patT, usage_p, sc_p))

  npatT, ngateT, nuse_p = pl.pallas_call(_phase_b, **_PHASE_B_KWARGS)(
      patT, gateT, usage_p, wcol, nusec, payT)

  (nswT,) = pl.pallas_call(_phase_w, **_PHASE_W_KWARGS)(swT, wcol, outersT)

  return (idx2.reshape(_B), sims, npatT.T,
          nswT.T.reshape(_C, _D, _D), ngateT.T,
          nuse_p.reshape(_CPAD)[:_C], load2.reshape(()))
